# baseline (speedup 1.0000x reference)
import jax
import jax.numpy as jnp
from jax.experimental import pallas as pl
from jax.experimental.pallas import tpu as pltpu


def _vq_body(mu_ref, dict_ref, dn2_ref, z_ref, ze_ref, oh_ref):
    tb = mu_ref.shape[0]
    num_codes, dict_size, dim_e = dict_ref.shape

    def dist_for(c):
        mu_c = mu_ref[:, c * dim_e:(c + 1) * dim_e]
        d_c = dict_ref[c]
        mn2 = jnp.sum(mu_c * mu_c, axis=1, keepdims=True)
        dn2 = dn2_ref[c]
        dot = jax.lax.dot_general(
            mu_c, d_c, (((1,), (1,)), ((), ())),
            preferred_element_type=jnp.float32)
        return mn2 + dn2[None, :] - 2.0 * dot

    ones_col = jnp.ones((dict_size, 128), dtype=jnp.float32)

    def process(c, dist):
        minv = jnp.min(dist, axis=1, keepdims=True)
        m = (dist == minv).astype(jnp.float32)
        oh_ref[:, c, :] = m
        ze = jax.lax.dot_general(
            m, dict_ref[c], (((1,), (0,)), ((), ())),
            preferred_element_type=jnp.float32)
        ze_ref[:, c * dim_e:(c + 1) * dim_e] = ze
        z_ref[:, c * dim_e:(c + 1) * dim_e] = ze
        cnt = jax.lax.dot_general(
            m, ones_col, (((1,), (0,)), ((), ())),
            preferred_element_type=jnp.float32)
        @pl.when(jnp.max(cnt) > 1.5)
        def _fix_ties():
            iota = jax.lax.broadcasted_iota(jnp.int32, (tb, dict_size), 1)
            idx = jnp.min(jnp.where(dist == minv, iota, dict_size), axis=1)
            oh = (iota == idx[:, None]).astype(jnp.float32)
            oh_ref[:, c, :] = oh
            zex = jax.lax.dot_general(
                oh, dict_ref[c], (((1,), (0,)), ((), ())),
                preferred_element_type=jnp.float32)
            ze_ref[:, c * dim_e:(c + 1) * dim_e] = zex
            z_ref[:, c * dim_e:(c + 1) * dim_e] = zex

    dist_prev = dist_for(0)
    for c in range(1, num_codes):
        dist_c = dist_for(c)
        process(c - 1, dist_prev)
        dist_prev = dist_c
    process(num_codes - 1, dist_prev)


def kernel(mu, dictionary):
    batch, feat = mu.shape
    num_codes, dict_size, dim_e = dictionary.shape
    dn2 = jnp.sum(dictionary ** 2, axis=-1)
    tb = 256
    grid = (batch // tb,)
    z, ze, oh = pl.pallas_call(
        _vq_body,
        grid=grid,
        in_specs=[
            pl.BlockSpec((tb, feat), lambda i: (i, 0)),
            pl.BlockSpec((num_codes, dict_size, dim_e), lambda i: (0, 0, 0)),
            pl.BlockSpec((num_codes, dict_size), lambda i: (0, 0)),
        ],
        out_specs=[
            pl.BlockSpec((tb, feat), lambda i: (i, 0)),
            pl.BlockSpec((tb, feat), lambda i: (i, 0)),
            pl.BlockSpec((tb, num_codes, dict_size), lambda i: (i, 0, 0)),
        ],
        out_shape=[
            jax.ShapeDtypeStruct((batch, feat), jnp.float32),
            jax.ShapeDtypeStruct((batch, feat), jnp.float32),
            jax.ShapeDtypeStruct((batch, num_codes, dict_size), jnp.float32),
        ],
        compiler_params=pltpu.CompilerParams(
            dimension_semantics=("parallel",)),
    )(mu, dictionary, dn2)
    return (z, ze, oh)


# speedup vs baseline: 1.2868x; 1.2868x over previous
import jax
import jax.numpy as jnp
from jax.experimental import pallas as pl
from jax.experimental.pallas import tpu as pltpu


def _vq_body(mu_ref, dict_ref, dn2_ref, z_ref, ze_ref, oh_ref):
    tb = mu_ref.shape[0]
    num_codes, dict_size, dim_e = dict_ref.shape

    def dist_for(c):
        mu_c = mu_ref[:, c * dim_e:(c + 1) * dim_e]
        d_c = dict_ref[c]
        mn2 = jnp.sum(mu_c * mu_c, axis=1, keepdims=True)
        dn2 = dn2_ref[c]
        dot = jax.lax.dot_general(
            mu_c, d_c, (((1,), (1,)), ((), ())),
            preferred_element_type=jnp.float32)
        return mn2 + dn2[None, :] - 2.0 * dot

    ones_col = jnp.ones((dict_size, 128), dtype=jnp.float32)

    def process(c, dist):
        minv = jnp.min(dist, axis=1, keepdims=True)
        m = (dist == minv).astype(jnp.float32)
        oh_ref[:, c, :] = m
        ze = jax.lax.dot_general(
            m, dict_ref[c], (((1,), (0,)), ((), ())),
            preferred_element_type=jnp.float32)
        ze_ref[:, c * dim_e:(c + 1) * dim_e] = ze
        z_ref[:, c * dim_e:(c + 1) * dim_e] = ze
        cnt = jax.lax.dot_general(
            m, ones_col, (((1,), (0,)), ((), ())),
            preferred_element_type=jnp.float32)
        return cnt

    dist_prev = dist_for(0)
    cmax = None
    for c in range(1, num_codes):
        dist_c = dist_for(c)
        cnt = process(c - 1, dist_prev)
        cmax = cnt if cmax is None else jnp.maximum(cmax, cnt)
        dist_prev = dist_c
    cmax = jnp.maximum(cmax, process(num_codes - 1, dist_prev))

    @pl.when(jnp.max(cmax) > 1.5)
    def _fix_ties():
        for c in range(num_codes):
            dist = dist_for(c)
            minv = jnp.min(dist, axis=1, keepdims=True)
            iota = jax.lax.broadcasted_iota(jnp.int32, (tb, dict_size), 1)
            idx = jnp.min(jnp.where(dist == minv, iota, dict_size), axis=1)
            oh = (iota == idx[:, None]).astype(jnp.float32)
            oh_ref[:, c, :] = oh
            zex = jax.lax.dot_general(
                oh, dict_ref[c], (((1,), (0,)), ((), ())),
                preferred_element_type=jnp.float32)
            ze_ref[:, c * dim_e:(c + 1) * dim_e] = zex
            z_ref[:, c * dim_e:(c + 1) * dim_e] = zex


def kernel(mu, dictionary):
    batch, feat = mu.shape
    num_codes, dict_size, dim_e = dictionary.shape
    dn2 = jnp.sum(dictionary ** 2, axis=-1)
    tb = 256
    grid = (batch // tb,)
    z, ze, oh = pl.pallas_call(
        _vq_body,
        grid=grid,
        in_specs=[
            pl.BlockSpec((tb, feat), lambda i: (i, 0)),
            pl.BlockSpec((num_codes, dict_size, dim_e), lambda i: (0, 0, 0)),
            pl.BlockSpec((num_codes, dict_size), lambda i: (0, 0)),
        ],
        out_specs=[
            pl.BlockSpec((tb, feat), lambda i: (i, 0)),
            pl.BlockSpec((tb, feat), lambda i: (i, 0)),
            pl.BlockSpec((tb, num_codes, dict_size), lambda i: (i, 0, 0)),
        ],
        out_shape=[
            jax.ShapeDtypeStruct((batch, feat), jnp.float32),
            jax.ShapeDtypeStruct((batch, feat), jnp.float32),
            jax.ShapeDtypeStruct((batch, num_codes, dict_size), jnp.float32),
        ],
        compiler_params=pltpu.CompilerParams(
            dimension_semantics=("parallel",)),
    )(mu, dictionary, dn2)
    return (z, ze, oh)
